# baseline (speedup 1.0000x reference)
import functools

import jax
import jax.numpy as jnp
from jax import lax
from jax.experimental import pallas as pl
from jax.experimental.pallas import tpu as pltpu
from jax.experimental.pallas import tpu_sc as plsc

_N = 10000
_E = 320000
_D = 128

_NC = 2
_NS = 16
_NW = _NC * _NS
_EPW = _E // _NW
_CH = 40
_NCH = _EPW // _CH
_RPT = 624
_REM = _N - _NS * _RPT


_NBUF = 5
_K = 3


def _sc_body(h_hbm, src_hbm, dst_hbm, zero_hbm, out_hbm, s1d, d1d, rows,
             acc, *sems):
    gsem = sems[:_NBUF]
    ssem = sems[_NBUF:]
    cid = lax.axis_index("c")
    sid = lax.axis_index("s")
    wid = sid * _NC + cid
    ebase = wid * _EPW

    z1 = pltpu.async_copy(zero_hbm, acc.at[pl.ds(sid * _RPT, _RPT)], gsem[0])
    i1 = pltpu.async_copy(src_hbm.at[pl.ds(ebase, _EPW)], s1d, gsem[1])
    i2 = pltpu.async_copy(dst_hbm.at[pl.ds(ebase, _EPW)], d1d, gsem[2])

    @pl.when(sid == _NS - 1)
    def _():
        pltpu.async_copy(zero_hbm.at[pl.ds(0, _REM)],
                         acc.at[pl.ds(_NS * _RPT, _REM)], ssem[0]).wait()

    z1.wait()
    i1.wait()
    i2.wait()
    plsc.subcore_barrier()

    def _gather(c, b):
        return pltpu.async_copy(
            h_hbm.at[s1d.at[pl.ds(c * _CH, _CH)]], rows.at[b], gsem[b])

    def _gwait(b):
        pltpu.make_async_copy(
            h_hbm.at[s1d.at[pl.ds(0, _CH)]], rows.at[b], gsem[b]).wait()

    def _swait(b):
        pltpu.make_async_copy(
            rows.at[b], acc.at[d1d.at[pl.ds(0, _CH)]], ssem[b]).wait()

    for b in range(_K):
        _gather(b, b)

    def body(c0, carry):
        for b in range(_NBUF):
            cc = c0 * _NBUF + b
            _gwait(b)
            pltpu.async_copy(
                rows.at[b], acc.at[d1d.at[pl.ds(cc * _CH, _CH)]], ssem[b],
                add=True)
            b2 = (b + _K) % _NBUF

            @pl.when(cc >= _NBUF - _K)
            def _():
                _swait(b2)

            @pl.when(cc + _K < _NCH)
            def _():
                _gather(cc + _K, b2)
        return carry

    lax.fori_loop(0, _NCH // _NBUF, body, 0)
    for c in range(_NCH - (_NBUF - _K), _NCH):
        _swait(c % _NBUF)
    plsc.subcore_barrier()
    pltpu.sync_copy(acc.at[pl.ds(sid * _RPT, _RPT)],
                    out_hbm.at[cid, pl.ds(sid * _RPT, _RPT)])

    @pl.when(sid == _NS - 1)
    def _():
        pltpu.sync_copy(acc.at[pl.ds(_NS * _RPT, _REM)],
                        out_hbm.at[cid, pl.ds(_NS * _RPT, _REM)])


_sc_segment_sum = pl.kernel(
    _sc_body,
    out_type=jax.ShapeDtypeStruct((_NC, _N, _D), jnp.float32),
    mesh=plsc.VectorSubcoreMesh(core_axis_name="c", subcore_axis_name="s"),
    scratch_types=[
        pltpu.VMEM((_EPW,), jnp.int32),
        pltpu.VMEM((_EPW,), jnp.int32),
        pltpu.VMEM((_NBUF, _CH, _D), jnp.float32),
        pltpu.VMEM_SHARED((_N, _D), jnp.float32),
    ] + [pltpu.SemaphoreType.DMA] * (2 * _NBUF),
)


def _bn_cols(x, g, b):
    m = jnp.mean(x, axis=0, keepdims=True)
    v = jnp.mean((x - m) * (x - m), axis=0, keepdims=True)
    return (x - m) * lax.rsqrt(v + 1e-5) * g + b


def _mlp_common(h_ref, p_ref, w0_ref, b0_ref, gm_ref, bm_ref, w1_ref, b1_ref):
    z = h_ref[...] + p_ref[0] + p_ref[1]
    t = jnp.dot(z, w0_ref[...], preferred_element_type=jnp.float32)
    t = _bn_cols(t + b0_ref[...], gm_ref[...], bm_ref[...])
    t = jnp.maximum(t, 0.0)
    o = jnp.dot(t, w1_ref[...], preferred_element_type=jnp.float32)
    return o + b1_ref[...]


def _mlp_mid_body(h_ref, p_ref, w0_ref, b0_ref, gm_ref, bm_ref, w1_ref,
                  b1_ref, go_ref, bo_ref, out_ref):
    o = _mlp_common(h_ref, p_ref, w0_ref, b0_ref, gm_ref, bm_ref, w1_ref,
                    b1_ref)
    out_ref[...] = _bn_cols(jnp.maximum(o, 0.0), go_ref[...], bo_ref[...])


def _mlp_last_body(h_ref, p_ref, w0_ref, b0_ref, gm_ref, bm_ref, w1_ref,
                   b1_ref, out_ref):
    o = _mlp_common(h_ref, p_ref, w0_ref, b0_ref, gm_ref, bm_ref, w1_ref,
                    b1_ref)
    mx = jnp.max(o, axis=1, keepdims=True)
    e = o - mx
    out_ref[...] = e - jnp.log(jnp.sum(jnp.exp(e), axis=1, keepdims=True))


_out_t = jax.ShapeDtypeStruct((_N, _D), jnp.float32)
_mlp_mid = pl.pallas_call(_mlp_mid_body, out_shape=_out_t)
_mlp_last = pl.pallas_call(_mlp_last_body, out_shape=_out_t)


def kernel(inputs, edge_index,
           W00, b00, gm0, bm0, W01, b01,
           W10, b10, gm1, bm1, W11, b11,
           W20, b20, gm2, bm2, W21, b21,
           go0, bo0, go1, bo1):
    src = edge_index[0]
    dst = edge_index[1]
    zero = jnp.zeros((_RPT, _D), jnp.float32)

    def row(v):
        return v.reshape(1, _D)

    h = inputs
    p = _sc_segment_sum(h, src, dst, zero)
    h = _mlp_mid(h, p, W00, row(b00), row(gm0), row(bm0), W01, row(b01),
                 row(go0), row(bo0))
    p = _sc_segment_sum(h, src, dst, zero)
    h = _mlp_mid(h, p, W10, row(b10), row(gm1), row(bm1), W11, row(b11),
                 row(go1), row(bo1))
    p = _sc_segment_sum(h, src, dst, zero)
    return _mlp_last(h, p, W20, row(b20), row(gm2), row(bm2), W21, row(b21))


# speedup vs baseline: 1.1247x; 1.1247x over previous
import functools

import jax
import jax.numpy as jnp
from jax import lax
from jax.experimental import pallas as pl
from jax.experimental.pallas import tpu as pltpu
from jax.experimental.pallas import tpu_sc as plsc

_N = 10000
_E = 320000
_D = 128

_NC = 2
_NS = 16
_NW = _NC * _NS
_EPW = _E // _NW
_CH = 40
_NCH = _EPW // _CH
_RPT = 624
_REM = _N - _NS * _RPT


_NBUF = 5
_K = 3


def _sc_body(h_hbm, src_hbm, dst_hbm, zero_hbm, out_hbm, s1d, d1d, rows,
             acc, *sems):
    gsem = sems[:_NBUF]
    ssem = sems[_NBUF:]
    cid = lax.axis_index("c")
    sid = lax.axis_index("s")
    wid = sid * _NC + cid
    ebase = wid * _EPW

    z1 = pltpu.async_copy(zero_hbm, acc.at[pl.ds(sid * _RPT, _RPT)], gsem[0])
    i1 = pltpu.async_copy(src_hbm.at[pl.ds(ebase, _EPW)], s1d, gsem[1])
    i2 = pltpu.async_copy(dst_hbm.at[pl.ds(ebase, _EPW)], d1d, gsem[2])

    @pl.when(sid == _NS - 1)
    def _():
        pltpu.async_copy(zero_hbm.at[pl.ds(0, _REM)],
                         acc.at[pl.ds(_NS * _RPT, _REM)], ssem[0]).wait()

    z1.wait()
    i1.wait()
    i2.wait()
    plsc.subcore_barrier()

    def _gather(c, b):
        return pltpu.async_copy(
            h_hbm.at[s1d.at[pl.ds(c * _CH, _CH)]], rows.at[b], gsem[b])

    def _gwait(b):
        pltpu.make_async_copy(
            h_hbm.at[s1d.at[pl.ds(0, _CH)]], rows.at[b], gsem[b]).wait()

    def _swait(b):
        pltpu.make_async_copy(
            rows.at[b], acc.at[d1d.at[pl.ds(0, _CH)]], ssem[b]).wait()

    for b in range(_NBUF):
        _gather(b, b)

    def body(c0, carry):
        for b in range(_NBUF):
            cc = c0 * _NBUF + b
            _gwait(b)
            pltpu.sync_copy(rows.at[b], acc.at[d1d.at[pl.ds(cc * _CH, _CH)]],
                            add=True)

            @pl.when(c0 < _NCH // _NBUF - 1)
            def _():
                _gather(cc + _NBUF, b)
        return carry

    lax.fori_loop(0, _NCH // _NBUF, body, 0)
    plsc.subcore_barrier()
    pltpu.sync_copy(acc.at[pl.ds(sid * _RPT, _RPT)],
                    out_hbm.at[cid, pl.ds(sid * _RPT, _RPT)])

    @pl.when(sid == _NS - 1)
    def _():
        pltpu.sync_copy(acc.at[pl.ds(_NS * _RPT, _REM)],
                        out_hbm.at[cid, pl.ds(_NS * _RPT, _REM)])


_sc_segment_sum = pl.kernel(
    _sc_body,
    out_type=jax.ShapeDtypeStruct((_NC, _N, _D), jnp.float32),
    mesh=plsc.VectorSubcoreMesh(core_axis_name="c", subcore_axis_name="s"),
    scratch_types=[
        pltpu.VMEM((_EPW,), jnp.int32),
        pltpu.VMEM((_EPW,), jnp.int32),
        pltpu.VMEM((_NBUF, _CH, _D), jnp.float32),
        pltpu.VMEM_SHARED((_N, _D), jnp.float32),
    ] + [pltpu.SemaphoreType.DMA] * (2 * _NBUF),
)


def _bn_cols(x, g, b):
    m = jnp.mean(x, axis=0, keepdims=True)
    v = jnp.mean((x - m) * (x - m), axis=0, keepdims=True)
    return (x - m) * lax.rsqrt(v + 1e-5) * g + b


def _mlp_common(h_ref, p_ref, w0_ref, b0_ref, gm_ref, bm_ref, w1_ref, b1_ref):
    z = h_ref[...] + p_ref[0] + p_ref[1]
    t = jnp.dot(z, w0_ref[...], preferred_element_type=jnp.float32)
    t = _bn_cols(t + b0_ref[...], gm_ref[...], bm_ref[...])
    t = jnp.maximum(t, 0.0)
    o = jnp.dot(t, w1_ref[...], preferred_element_type=jnp.float32)
    return o + b1_ref[...]


def _mlp_mid_body(h_ref, p_ref, w0_ref, b0_ref, gm_ref, bm_ref, w1_ref,
                  b1_ref, go_ref, bo_ref, out_ref):
    o = _mlp_common(h_ref, p_ref, w0_ref, b0_ref, gm_ref, bm_ref, w1_ref,
                    b1_ref)
    out_ref[...] = _bn_cols(jnp.maximum(o, 0.0), go_ref[...], bo_ref[...])


def _mlp_last_body(h_ref, p_ref, w0_ref, b0_ref, gm_ref, bm_ref, w1_ref,
                   b1_ref, out_ref):
    o = _mlp_common(h_ref, p_ref, w0_ref, b0_ref, gm_ref, bm_ref, w1_ref,
                    b1_ref)
    mx = jnp.max(o, axis=1, keepdims=True)
    e = o - mx
    out_ref[...] = e - jnp.log(jnp.sum(jnp.exp(e), axis=1, keepdims=True))


_out_t = jax.ShapeDtypeStruct((_N, _D), jnp.float32)
_mlp_mid = pl.pallas_call(_mlp_mid_body, out_shape=_out_t)
_mlp_last = pl.pallas_call(_mlp_last_body, out_shape=_out_t)


def kernel(inputs, edge_index,
           W00, b00, gm0, bm0, W01, b01,
           W10, b10, gm1, bm1, W11, b11,
           W20, b20, gm2, bm2, W21, b21,
           go0, bo0, go1, bo1):
    src = edge_index[0]
    dst = edge_index[1]
    zero = jnp.zeros((_RPT, _D), jnp.float32)

    def row(v):
        return v.reshape(1, _D)

    h = inputs
    p = _sc_segment_sum(h, src, dst, zero)
    h = _mlp_mid(h, p, W00, row(b00), row(gm0), row(bm0), W01, row(b01),
                 row(go0), row(bo0))
    p = _sc_segment_sum(h, src, dst, zero)
    h = _mlp_mid(h, p, W10, row(b10), row(gm1), row(bm1), W11, row(b11),
                 row(go1), row(bo1))
    p = _sc_segment_sum(h, src, dst, zero)
    return _mlp_last(h, p, W20, row(b20), row(gm2), row(bm2), W21, row(b21))


# speedup vs baseline: 1.1273x; 1.0023x over previous
import functools

import jax
import jax.numpy as jnp
from jax import lax
from jax.experimental import pallas as pl
from jax.experimental.pallas import tpu as pltpu
from jax.experimental.pallas import tpu_sc as plsc

_N = 10000
_E = 320000
_D = 128

_NC = 2
_NS = 16
_NW = _NC * _NS
_EPW = _E // _NW
_CH = 80
_NCH = _EPW // _CH
_RPT = 624
_REM = _N - _NS * _RPT


_NBUF = 4
_NIB = 2 * _NBUF


def _sc_body(h_hbm, src_hbm, dst_hbm, zero_hbm, out_hbm, sidx, didx, rows,
             acc, *sems):
    gsem = sems[:_NBUF]
    isem = sems[_NBUF:]
    cid = lax.axis_index("c")
    sid = lax.axis_index("s")
    wid = sid * _NC + cid
    ebase = wid * _EPW

    def _idx_start(c, slot):
        base = ebase + c * _CH
        pltpu.async_copy(src_hbm.at[pl.ds(base, _CH)], sidx.at[slot],
                         isem[slot])
        pltpu.async_copy(dst_hbm.at[pl.ds(base, _CH)], didx.at[slot],
                         isem[slot])

    def _idx_wait(slot):
        pltpu.make_async_copy(src_hbm.at[pl.ds(0, _CH)], sidx.at[slot],
                              isem[slot]).wait()
        pltpu.make_async_copy(dst_hbm.at[pl.ds(0, _CH)], didx.at[slot],
                              isem[slot]).wait()

    def _gather(c_slot, b):
        return pltpu.async_copy(
            h_hbm.at[sidx.at[c_slot]], rows.at[b], gsem[b])

    def _gwait(b):
        pltpu.make_async_copy(
            h_hbm.at[sidx.at[0]], rows.at[b], gsem[b]).wait()

    z1 = pltpu.async_copy(zero_hbm, acc.at[pl.ds(sid * _RPT, _RPT)], gsem[0])
    for s in range(_NIB):
        _idx_start(s, s)

    @pl.when(sid == _NS - 1)
    def _():
        pltpu.async_copy(zero_hbm.at[pl.ds(0, _REM)],
                         acc.at[pl.ds(_NS * _RPT, _REM)], gsem[1]).wait()

    z1.wait()
    plsc.subcore_barrier()

    for b in range(_NBUF):
        _idx_wait(b)
        _gather(b, b)

    def _visit(cc, s):
        b = s % _NBUF
        _gwait(b)
        pltpu.sync_copy(rows.at[b], acc.at[didx.at[s]], add=True)

        @pl.when(cc + _NBUF < _NCH)
        def _():
            _idx_wait((s + _NBUF) % _NIB)
            _gather((s + _NBUF) % _NIB, b)

        @pl.when(cc + _NIB < _NCH)
        def _():
            _idx_start(cc + _NIB, s)

    def body(c0, carry):
        for s in range(_NIB):
            _visit(c0 * _NIB + s, s)
        return carry

    _NFULL = (_NCH // _NIB) * _NIB
    lax.fori_loop(0, _NCH // _NIB, body, 0)
    for t in range(_NCH - _NFULL):
        _visit(_NFULL + t, t)
    plsc.subcore_barrier()
    pltpu.sync_copy(acc.at[pl.ds(sid * _RPT, _RPT)],
                    out_hbm.at[cid, pl.ds(sid * _RPT, _RPT)])

    @pl.when(sid == _NS - 1)
    def _():
        pltpu.sync_copy(acc.at[pl.ds(_NS * _RPT, _REM)],
                        out_hbm.at[cid, pl.ds(_NS * _RPT, _REM)])


_sc_segment_sum = pl.kernel(
    _sc_body,
    out_type=jax.ShapeDtypeStruct((_NC, _N, _D), jnp.float32),
    mesh=plsc.VectorSubcoreMesh(core_axis_name="c", subcore_axis_name="s"),
    scratch_types=[
        pltpu.VMEM((_NIB, _CH), jnp.int32),
        pltpu.VMEM((_NIB, _CH), jnp.int32),
        pltpu.VMEM((_NBUF, _CH, _D), jnp.float32),
        pltpu.VMEM_SHARED((_N, _D), jnp.float32),
    ] + [pltpu.SemaphoreType.DMA] * (_NBUF + _NIB),
)


def _bn_cols(x, g, b):
    m = jnp.mean(x, axis=0, keepdims=True)
    v = jnp.mean((x - m) * (x - m), axis=0, keepdims=True)
    return (x - m) * lax.rsqrt(v + 1e-5) * g + b


def _mlp_common(h_ref, p_ref, w0_ref, b0_ref, gm_ref, bm_ref, w1_ref, b1_ref):
    z = h_ref[...] + p_ref[0] + p_ref[1]
    t = jnp.dot(z, w0_ref[...], preferred_element_type=jnp.float32)
    t = _bn_cols(t + b0_ref[...], gm_ref[...], bm_ref[...])
    t = jnp.maximum(t, 0.0)
    o = jnp.dot(t, w1_ref[...], preferred_element_type=jnp.float32)
    return o + b1_ref[...]


def _mlp_mid_body(h_ref, p_ref, w0_ref, b0_ref, gm_ref, bm_ref, w1_ref,
                  b1_ref, go_ref, bo_ref, out_ref):
    o = _mlp_common(h_ref, p_ref, w0_ref, b0_ref, gm_ref, bm_ref, w1_ref,
                    b1_ref)
    out_ref[...] = _bn_cols(jnp.maximum(o, 0.0), go_ref[...], bo_ref[...])


def _mlp_last_body(h_ref, p_ref, w0_ref, b0_ref, gm_ref, bm_ref, w1_ref,
                   b1_ref, out_ref):
    o = _mlp_common(h_ref, p_ref, w0_ref, b0_ref, gm_ref, bm_ref, w1_ref,
                    b1_ref)
    mx = jnp.max(o, axis=1, keepdims=True)
    e = o - mx
    out_ref[...] = e - jnp.log(jnp.sum(jnp.exp(e), axis=1, keepdims=True))


_out_t = jax.ShapeDtypeStruct((_N, _D), jnp.float32)
_mlp_mid = pl.pallas_call(_mlp_mid_body, out_shape=_out_t)
_mlp_last = pl.pallas_call(_mlp_last_body, out_shape=_out_t)


def kernel(inputs, edge_index,
           W00, b00, gm0, bm0, W01, b01,
           W10, b10, gm1, bm1, W11, b11,
           W20, b20, gm2, bm2, W21, b21,
           go0, bo0, go1, bo1):
    src = edge_index[0]
    dst = edge_index[1]
    zero = jnp.zeros((_RPT, _D), jnp.float32)

    def row(v):
        return v.reshape(1, _D)

    h = inputs
    p = _sc_segment_sum(h, src, dst, zero)
    h = _mlp_mid(h, p, W00, row(b00), row(gm0), row(bm0), W01, row(b01),
                 row(go0), row(bo0))
    p = _sc_segment_sum(h, src, dst, zero)
    h = _mlp_mid(h, p, W10, row(b10), row(gm1), row(bm1), W11, row(b11),
                 row(go1), row(bo1))
    p = _sc_segment_sum(h, src, dst, zero)
    return _mlp_last(h, p, W20, row(b20), row(gm2), row(bm2), W21, row(b21))
